# initial kernel scaffold (unmeasured)
import jax
import jax.numpy as jnp
from jax import lax
from jax.experimental import pallas as pl
from jax.experimental.pallas import tpu as pltpu


def kernel(
    x,
):
    def body(*refs):
        pass

    out_shape = jax.ShapeDtypeStruct(..., jnp.float32)
    return pl.pallas_call(body, out_shape=out_shape)(...)


# baseline (device time: 333481 ns/iter reference)
import jax
import jax.numpy as jnp
from jax import lax
from jax.experimental import pallas as pl
from jax.experimental.pallas import tpu as pltpu

N_DEV = 4


def kernel(x):
    x = x.astype(jnp.bfloat16)
    m_per, n = x.shape
    half = m_per // 2

    def body(x_ref, out_ref, copy_sem, cw_send, cw_recv, ccw_send, ccw_recv):
        my = lax.axis_index("i")
        left = lax.rem(my + N_DEV - 1, N_DEV)
        right = lax.rem(my + 1, N_DEV)

        local = pltpu.make_async_copy(
            x_ref, out_ref.at[pl.ds(my * m_per, m_per), :], copy_sem
        )
        local.start()

        barrier = pltpu.get_barrier_semaphore()
        for nbr in (left, right):
            pl.semaphore_signal(
                barrier, inc=1, device_id=(nbr,),
                device_id_type=pl.DeviceIdType.MESH,
            )
        pl.semaphore_wait(barrier, 2)

        for h in range(N_DEV - 1):
            o_cw = lax.rem(my + N_DEV - h, N_DEV)
            o_ccw = lax.rem(my + h, N_DEV)
            if h == 0:
                src_cw = x_ref.at[pl.ds(0, half), :]
                src_ccw = x_ref.at[pl.ds(half, half), :]
            else:
                src_cw = out_ref.at[pl.ds(o_cw * m_per, half), :]
                src_ccw = out_ref.at[pl.ds(o_ccw * m_per + half, half), :]

            rdma_cw = pltpu.make_async_remote_copy(
                src_ref=src_cw,
                dst_ref=out_ref.at[pl.ds(o_cw * m_per, half), :],
                send_sem=cw_send.at[h],
                recv_sem=cw_recv.at[h],
                device_id=(right,),
                device_id_type=pl.DeviceIdType.MESH,
            )
            rdma_ccw = pltpu.make_async_remote_copy(
                src_ref=src_ccw,
                dst_ref=out_ref.at[pl.ds(o_ccw * m_per + half, half), :],
                send_sem=ccw_send.at[h],
                recv_sem=ccw_recv.at[h],
                device_id=(left,),
                device_id_type=pl.DeviceIdType.MESH,
            )
            rdma_cw.start()
            rdma_ccw.start()
            rdma_cw.wait()
            rdma_ccw.wait()

        local.wait()

    return pl.pallas_call(
        body,
        out_shape=jax.ShapeDtypeStruct((N_DEV * m_per, n), jnp.bfloat16),
        in_specs=[pl.BlockSpec(memory_space=pl.ANY)],
        out_specs=pl.BlockSpec(memory_space=pl.ANY),
        scratch_shapes=[
            pltpu.SemaphoreType.DMA,
            pltpu.SemaphoreType.DMA((N_DEV - 1,)),
            pltpu.SemaphoreType.DMA((N_DEV - 1,)),
            pltpu.SemaphoreType.DMA((N_DEV - 1,)),
            pltpu.SemaphoreType.DMA((N_DEV - 1,)),
        ],
        compiler_params=pltpu.CompilerParams(collective_id=0),
    )(x)


# device time: 331968 ns/iter; 1.0046x vs baseline; 1.0046x over previous
import jax
import jax.numpy as jnp
from jax import lax
from jax.experimental import pallas as pl
from jax.experimental.pallas import tpu as pltpu

N_DEV = 4


def kernel(x):
    m_per, n = x.shape
    half = m_per // 2

    def body(x_ref, out_ref, vf32, vbf, in_sem, copy_sem,
             cw_send, cw_recv, ccw_send, ccw_recv):
        my = lax.axis_index("i")
        left = lax.rem(my + N_DEV - 1, N_DEV)
        right = lax.rem(my + 1, N_DEV)

        cp_in = pltpu.make_async_copy(x_ref, vf32, in_sem)
        cp_in.start()

        barrier = pltpu.get_barrier_semaphore()
        for nbr in (left, right):
            pl.semaphore_signal(
                barrier, inc=1, device_id=(nbr,),
                device_id_type=pl.DeviceIdType.MESH,
            )
        pl.semaphore_wait(barrier, 2)
        cp_in.wait()

        def make_rdma(h, direction):
            if direction == 0:
                origin = lax.rem(my + N_DEV - h, N_DEV)
                row0, nbr = origin * m_per, right
                send_s, recv_s = cw_send, cw_recv
                src = vbf.at[pl.ds(0, half), :] if h == 0 else (
                    out_ref.at[pl.ds(row0, half), :])
            else:
                origin = lax.rem(my + h, N_DEV)
                row0, nbr = origin * m_per + half, left
                send_s, recv_s = ccw_send, ccw_recv
                src = vbf.at[pl.ds(half, half), :] if h == 0 else (
                    out_ref.at[pl.ds(row0, half), :])
            return pltpu.make_async_remote_copy(
                src_ref=src,
                dst_ref=out_ref.at[pl.ds(row0, half), :],
                send_sem=send_s.at[h],
                recv_sem=recv_s.at[h],
                device_id=(nbr,),
                device_id_type=pl.DeviceIdType.MESH,
            )

        vbf[pl.ds(0, half), :] = vf32[pl.ds(0, half), :].astype(jnp.bfloat16)
        rdmas = [[None, None] for _ in range(N_DEV - 1)]
        rdmas[0][0] = make_rdma(0, 0)
        rdmas[0][0].start()
        vbf[pl.ds(half, half), :] = vf32[pl.ds(half, half), :].astype(
            jnp.bfloat16)
        rdmas[0][1] = make_rdma(0, 1)
        rdmas[0][1].start()

        cp_out = pltpu.make_async_copy(
            vbf, out_ref.at[pl.ds(my * m_per, m_per), :], copy_sem
        )
        cp_out.start()

        for h in range(1, N_DEV - 1):
            for d in (0, 1):
                rdmas[h - 1][d].wait_recv()
                rdmas[h][d] = make_rdma(h, d)
                rdmas[h][d].start()
        for d in (0, 1):
            rdmas[N_DEV - 2][d].wait_recv()

        for h in range(N_DEV - 1):
            for d in (0, 1):
                rdmas[h][d].wait_send()
        cp_out.wait()

    return pl.pallas_call(
        body,
        out_shape=jax.ShapeDtypeStruct((N_DEV * m_per, n), jnp.bfloat16),
        in_specs=[pl.BlockSpec(memory_space=pl.ANY)],
        out_specs=pl.BlockSpec(memory_space=pl.ANY),
        scratch_shapes=[
            pltpu.VMEM((m_per, n), jnp.float32),
            pltpu.VMEM((m_per, n), jnp.bfloat16),
            pltpu.SemaphoreType.DMA,
            pltpu.SemaphoreType.DMA,
            pltpu.SemaphoreType.DMA((N_DEV - 1,)),
            pltpu.SemaphoreType.DMA((N_DEV - 1,)),
            pltpu.SemaphoreType.DMA((N_DEV - 1,)),
            pltpu.SemaphoreType.DMA((N_DEV - 1,)),
        ],
        compiler_params=pltpu.CompilerParams(
            collective_id=0,
            vmem_limit_bytes=96 * 1024 * 1024,
        ),
    )(x)


# device time: 319895 ns/iter; 1.0425x vs baseline; 1.0377x over previous
import jax
import jax.numpy as jnp
from jax import lax
from jax.experimental import pallas as pl
from jax.experimental.pallas import tpu as pltpu

N_DEV = 4
P = 4


def kernel(x):
    m_per, n = x.shape
    half = m_per // 2
    rows = half // P

    def body(x_ref, out_ref, vf32, vbf, in_sems, copy_sem,
             cw_send, cw_recv, ccw_send, ccw_recv):
        my = lax.axis_index("i")
        left = lax.rem(my + N_DEV - 1, N_DEV)
        right = lax.rem(my + 1, N_DEV)

        stage = []
        for p in range(2 * P):
            d, q = p % 2, p // 2
            r0 = q * rows + (0 if d == 0 else half)
            cp = pltpu.make_async_copy(
                x_ref.at[pl.ds(r0, rows), :],
                vf32.at[pl.ds(r0, rows), :],
                in_sems.at[p],
            )
            cp.start()
            stage.append((cp, d, q, r0))

        barrier = pltpu.get_barrier_semaphore()
        for nbr in (left, right):
            pl.semaphore_signal(
                barrier, inc=1, device_id=(nbr,),
                device_id_type=pl.DeviceIdType.MESH,
            )
        pl.semaphore_wait(barrier, 2)

        def make_rdma(h, d, q):
            if d == 0:
                origin = lax.rem(my + N_DEV - h, N_DEV)
                base, nbr = origin * m_per, right
                send_s, recv_s = cw_send, cw_recv
                loc = q * rows
            else:
                origin = lax.rem(my + h, N_DEV)
                base, nbr = origin * m_per + half, left
                send_s, recv_s = ccw_send, ccw_recv
                loc = half + q * rows
            row0 = base + q * rows
            src = (vbf.at[pl.ds(loc, rows), :] if h == 0
                   else out_ref.at[pl.ds(row0, rows), :])
            return pltpu.make_async_remote_copy(
                src_ref=src,
                dst_ref=out_ref.at[pl.ds(row0, rows), :],
                send_sem=send_s.at[h, q],
                recv_sem=recv_s.at[h, q],
                device_id=(nbr,),
                device_id_type=pl.DeviceIdType.MESH,
            )

        rdmas = [[[None] * P for _ in range(2)] for _ in range(N_DEV - 1)]

        for cp, d, q, r0 in stage:
            cp.wait()
            vbf[pl.ds(r0, rows), :] = vf32[pl.ds(r0, rows), :].astype(
                jnp.bfloat16)
            r = make_rdma(0, d, q)
            r.start()
            rdmas[0][d][q] = r

        cp_out = pltpu.make_async_copy(
            vbf, out_ref.at[pl.ds(my * m_per, m_per), :], copy_sem
        )
        cp_out.start()

        for h in range(1, N_DEV - 1):
            for q in range(P):
                for d in (0, 1):
                    rdmas[h - 1][d][q].wait_recv()
                    r = make_rdma(h, d, q)
                    r.start()
                    rdmas[h][d][q] = r
        for q in range(P):
            for d in (0, 1):
                rdmas[N_DEV - 2][d][q].wait_recv()

        for h in range(N_DEV - 1):
            for d in (0, 1):
                for q in range(P):
                    rdmas[h][d][q].wait_send()
        cp_out.wait()

    return pl.pallas_call(
        body,
        out_shape=jax.ShapeDtypeStruct((N_DEV * m_per, n), jnp.bfloat16),
        in_specs=[pl.BlockSpec(memory_space=pl.ANY)],
        out_specs=pl.BlockSpec(memory_space=pl.ANY),
        scratch_shapes=[
            pltpu.VMEM((m_per, n), jnp.float32),
            pltpu.VMEM((m_per, n), jnp.bfloat16),
            pltpu.SemaphoreType.DMA((2 * P,)),
            pltpu.SemaphoreType.DMA,
            pltpu.SemaphoreType.DMA((N_DEV - 1, P)),
            pltpu.SemaphoreType.DMA((N_DEV - 1, P)),
            pltpu.SemaphoreType.DMA((N_DEV - 1, P)),
            pltpu.SemaphoreType.DMA((N_DEV - 1, P)),
        ],
        compiler_params=pltpu.CompilerParams(
            collective_id=0,
            vmem_limit_bytes=96 * 1024 * 1024,
        ),
    )(x)


# device time: 318821 ns/iter; 1.0460x vs baseline; 1.0034x over previous
import jax
import jax.numpy as jnp
from jax import lax
from jax.experimental import pallas as pl
from jax.experimental.pallas import tpu as pltpu

N_DEV = 4
P = 8


def kernel(x):
    m_per, n = x.shape
    half = m_per // 2
    rows = half // P

    def body(x_ref, out_ref, vf32, vbf, in_sems, copy_sem,
             cw_send, cw_recv, ccw_send, ccw_recv):
        my = lax.axis_index("i")
        left = lax.rem(my + N_DEV - 1, N_DEV)
        right = lax.rem(my + 1, N_DEV)

        stage = []
        for p in range(2 * P):
            d, q = p % 2, p // 2
            r0 = q * rows + (0 if d == 0 else half)
            cp = pltpu.make_async_copy(
                x_ref.at[pl.ds(r0, rows), :],
                vf32.at[pl.ds(r0, rows), :],
                in_sems.at[p],
            )
            cp.start()
            stage.append((cp, d, q, r0))

        barrier = pltpu.get_barrier_semaphore()
        for nbr in (left, right):
            pl.semaphore_signal(
                barrier, inc=1, device_id=(nbr,),
                device_id_type=pl.DeviceIdType.MESH,
            )
        pl.semaphore_wait(barrier, 2)

        def make_rdma(h, d, q):
            if d == 0:
                origin = lax.rem(my + N_DEV - h, N_DEV)
                base, nbr = origin * m_per, right
                send_s, recv_s = cw_send, cw_recv
                loc = q * rows
            else:
                origin = lax.rem(my + h, N_DEV)
                base, nbr = origin * m_per + half, left
                send_s, recv_s = ccw_send, ccw_recv
                loc = half + q * rows
            row0 = base + q * rows
            src = (vbf.at[pl.ds(loc, rows), :] if h == 0
                   else out_ref.at[pl.ds(row0, rows), :])
            return pltpu.make_async_remote_copy(
                src_ref=src,
                dst_ref=out_ref.at[pl.ds(row0, rows), :],
                send_sem=send_s.at[h, q],
                recv_sem=recv_s.at[h, q],
                device_id=(nbr,),
                device_id_type=pl.DeviceIdType.MESH,
            )

        rdmas = [[[None] * P for _ in range(2)] for _ in range(N_DEV - 1)]

        for cp, d, q, r0 in stage:
            cp.wait()
            vbf[pl.ds(r0, rows), :] = vf32[pl.ds(r0, rows), :].astype(
                jnp.bfloat16)
            r = make_rdma(0, d, q)
            r.start()
            rdmas[0][d][q] = r

        cp_out = pltpu.make_async_copy(
            vbf, out_ref.at[pl.ds(my * m_per, m_per), :], copy_sem
        )
        cp_out.start()

        for h in range(1, N_DEV - 1):
            for q in range(P):
                for d in (0, 1):
                    rdmas[h - 1][d][q].wait_recv()
                    r = make_rdma(h, d, q)
                    r.start()
                    rdmas[h][d][q] = r
        for q in range(P):
            for d in (0, 1):
                rdmas[N_DEV - 2][d][q].wait_recv()

        for h in range(N_DEV - 1):
            for d in (0, 1):
                for q in range(P):
                    rdmas[h][d][q].wait_send()
        cp_out.wait()

    return pl.pallas_call(
        body,
        out_shape=jax.ShapeDtypeStruct((N_DEV * m_per, n), jnp.bfloat16),
        in_specs=[pl.BlockSpec(memory_space=pl.ANY)],
        out_specs=pl.BlockSpec(memory_space=pl.ANY),
        scratch_shapes=[
            pltpu.VMEM((m_per, n), jnp.float32),
            pltpu.VMEM((m_per, n), jnp.bfloat16),
            pltpu.SemaphoreType.DMA((2 * P,)),
            pltpu.SemaphoreType.DMA,
            pltpu.SemaphoreType.DMA((N_DEV - 1, P)),
            pltpu.SemaphoreType.DMA((N_DEV - 1, P)),
            pltpu.SemaphoreType.DMA((N_DEV - 1, P)),
            pltpu.SemaphoreType.DMA((N_DEV - 1, P)),
        ],
        compiler_params=pltpu.CompilerParams(
            collective_id=0,
            vmem_limit_bytes=96 * 1024 * 1024,
        ),
    )(x)
